# baseline (speedup 1.0000x reference)
import functools

import jax
import jax.numpy as jnp
from jax import lax
from jax.experimental import pallas as pl
from jax.experimental.pallas import tpu as pltpu
from jax.experimental.pallas import tpu_sc as plsc

N_ENTITIES = 1000000
N_RELATIONS = 1000
EMBED_DIM = 64
BATCH = 16384

NUM_WORKERS = 32
B_PER_W = BATCH // NUM_WORKERS
CHUNK = 128
N_CHUNKS = B_PER_W // CHUNK
GROUPS_PER_CHUNK = CHUNK // 16


def _sc_body(heads_hbm, rels_hbm, tails_hbm, etab_hbm, rtab_hbm, out_hbm,
             h_idx, r_idx, t_idx, h_rows, r_rows, t_rows, out_v,
             sem0, sem1, sem2, sem3):
    sems = (sem0, sem1, sem2, sem3)
    wid = lax.axis_index("s") * 2 + lax.axis_index("c")
    base = wid * B_PER_W

    for j in range(N_CHUNKS):
        off = base + j * CHUNK
        pltpu.sync_copy(heads_hbm.at[pl.ds(off, CHUNK)], h_idx.at[j])
        pltpu.sync_copy(rels_hbm.at[pl.ds(off, CHUNK)], r_idx.at[j])
        pltpu.sync_copy(tails_hbm.at[pl.ds(off, CHUNK)], t_idx.at[j])

    handles = []
    for j in range(N_CHUNKS):
        rows = pl.ds(j * CHUNK, CHUNK)
        handles.append((
            pltpu.async_copy(etab_hbm.at[h_idx.at[j]], h_rows.at[rows], sems[j]),
            pltpu.async_copy(rtab_hbm.at[r_idx.at[j]], r_rows.at[rows], sems[j]),
            pltpu.async_copy(etab_hbm.at[t_idx.at[j]], t_rows.at[rows], sems[j]),
        ))

    lane = lax.iota(jnp.int32, 16)

    for j in range(N_CHUNKS):
        for h in handles[j]:
            h.wait()

        def group_body(g, _, j=j):
            row0 = j * CHUNK + g * 16
            row_idx = lane + row0
            acc = jnp.zeros((16,), jnp.float32)
            for d in range(EMBED_DIM):
                col = jnp.full((16,), d, jnp.int32)
                vh = plsc.load_gather(h_rows, [row_idx, col])
                vr = plsc.load_gather(r_rows, [row_idx, col])
                vt = plsc.load_gather(t_rows, [row_idx, col])
                s = (vh + vr) - vt
                acc = acc + s * s
            out_v[pl.ds(row0, 16)] = -acc
            return 0

        lax.fori_loop(0, GROUPS_PER_CHUNK, group_body, 0)

    pltpu.sync_copy(out_v, out_hbm.at[pl.ds(base, B_PER_W)])


@jax.jit
def _score(heads, relations, tails, entity_table, relation_table):
    mesh = plsc.VectorSubcoreMesh(core_axis_name="c", subcore_axis_name="s")
    f = functools.partial(
        pl.kernel,
        mesh=mesh,
        compiler_params=pltpu.CompilerParams(
            needs_layout_passes=False, use_tc_tiling_on_sc=False),
        out_type=jax.ShapeDtypeStruct((BATCH,), jnp.float32),
        scratch_types=[
            pltpu.VMEM((N_CHUNKS, CHUNK), jnp.int32),
            pltpu.VMEM((N_CHUNKS, CHUNK), jnp.int32),
            pltpu.VMEM((N_CHUNKS, CHUNK), jnp.int32),
            pltpu.VMEM((B_PER_W, EMBED_DIM), jnp.float32),
            pltpu.VMEM((B_PER_W, EMBED_DIM), jnp.float32),
            pltpu.VMEM((B_PER_W, EMBED_DIM), jnp.float32),
            pltpu.VMEM((B_PER_W,), jnp.float32),
            pltpu.SemaphoreType.DMA,
            pltpu.SemaphoreType.DMA,
            pltpu.SemaphoreType.DMA,
            pltpu.SemaphoreType.DMA,
        ],
    )(_sc_body)
    return f(heads, relations, tails, entity_table, relation_table)


def kernel(heads, relations, tails, entity_table, relation_table):
    return _score(heads.astype(jnp.int32), relations.astype(jnp.int32),
                  tails.astype(jnp.int32), entity_table, relation_table)


# speedup vs baseline: 1.3671x; 1.3671x over previous
import functools

import jax
import jax.numpy as jnp
from jax import lax
from jax.experimental import pallas as pl
from jax.experimental.pallas import tpu as pltpu
from jax.experimental.pallas import tpu_sc as plsc

N_ENTITIES = 1000000
N_RELATIONS = 1000
EMBED_DIM = 64
BATCH = 16384

NUM_WORKERS = 32
B_PER_W = BATCH // NUM_WORKERS
CH = 16
N_CHUNKS = B_PER_W // CH


def _sc_body(heads_hbm, rels_hbm, tails_hbm, etab_hbm, rtab_hbm, out_hbm,
             h_idx, r_idx, t_idx, blocks, out_v, sem):
    wid = lax.axis_index("s") * 2 + lax.axis_index("c")
    base = wid * B_PER_W

    pltpu.sync_copy(heads_hbm.at[pl.ds(base, B_PER_W)], h_idx)
    pltpu.sync_copy(rels_hbm.at[pl.ds(base, B_PER_W)], r_idx)
    pltpu.sync_copy(tails_hbm.at[pl.ds(base, B_PER_W)], t_idx)

    lane = lax.iota(jnp.int32, 16)

    def chunk_body(c, _):
        hv = h_idx[pl.ds(c * CH, 16)]
        tv = t_idx[pl.ds(c * CH, 16)]
        rv = r_idx[pl.ds(c * CH, 16)]
        hg = lax.shift_right_logical(hv, 3) * 8
        tg = lax.shift_right_logical(tv, 3) * 8
        rg = lax.shift_right_logical(rv, 3) * 8
        handles = []
        for i in range(CH):
            handles.append(pltpu.async_copy(
                etab_hbm.at[pl.ds(pl.multiple_of(hg[i], 8), 8), :],
                blocks.at[i], sem))
            handles.append(pltpu.async_copy(
                etab_hbm.at[pl.ds(pl.multiple_of(tg[i], 8), 8), :],
                blocks.at[CH + i], sem))
            handles.append(pltpu.async_copy(
                rtab_hbm.at[pl.ds(pl.multiple_of(rg[i], 8), 8), :],
                blocks.at[2 * CH + i], sem))
        for h in handles:
            h.wait()

        hs = lax.bitwise_and(hv, 7)
        ts = lax.bitwise_and(tv, 7)
        rs = lax.bitwise_and(rv, 7)
        ent_h = lane
        ent_t = lane + CH
        ent_r = lane + 2 * CH
        acc = jnp.zeros((16,), jnp.float32)
        for d in range(EMBED_DIM):
            dsplat = jnp.full((16,), d, jnp.int32)
            vh = plsc.load_gather(blocks, [ent_h, hs, dsplat])
            vt = plsc.load_gather(blocks, [ent_t, ts, dsplat])
            vr = plsc.load_gather(blocks, [ent_r, rs, dsplat])
            s = (vh + vr) - vt
            acc = acc + s * s
        out_v[pl.ds(c * CH, 16)] = -acc
        return 0

    lax.fori_loop(0, N_CHUNKS, chunk_body, 0)

    pltpu.sync_copy(out_v, out_hbm.at[pl.ds(base, B_PER_W)])


@jax.jit
def _score(heads, relations, tails, etab, rtab):
    mesh = plsc.VectorSubcoreMesh(core_axis_name="c", subcore_axis_name="s")
    f = functools.partial(
        pl.kernel,
        mesh=mesh,
        compiler_params=pltpu.CompilerParams(
            needs_layout_passes=False, use_tc_tiling_on_sc=True),
        out_type=jax.ShapeDtypeStruct((BATCH,), jnp.float32),
        scratch_types=[
            pltpu.VMEM((B_PER_W,), jnp.int32),
            pltpu.VMEM((B_PER_W,), jnp.int32),
            pltpu.VMEM((B_PER_W,), jnp.int32),
            pltpu.VMEM((3 * CH, 8, EMBED_DIM), jnp.float32),
            pltpu.VMEM((B_PER_W,), jnp.float32),
            pltpu.SemaphoreType.DMA,
        ],
    )(_sc_body)
    return f(heads, relations, tails, etab, rtab)


def kernel(heads, relations, tails, entity_table, relation_table):
    return _score(heads.astype(jnp.int32), relations.astype(jnp.int32),
                  tails.astype(jnp.int32), entity_table, relation_table)
